# baseline (speedup 1.0000x reference)
import functools

import jax
import jax.numpy as jnp
from jax import lax
from jax.experimental import pallas as pl
from jax.experimental.pallas import tpu as pltpu
from jax.experimental.pallas import tpu_sc as plsc

NC, NS, L = 2, 16, 16
NW = NC * NS

D = 128
K = 50
KG = 100
RBLK = 1000


def _tanh_f32(x):
    x = jnp.clip(x, -7.90531110763549805, 7.90531110763549805)
    x2 = x * x
    p = 2.00018790482477e-13 + x2 * -2.76076847742355e-16
    p = -8.60467152213735e-11 + x2 * p
    p = 5.12229709037114e-08 + x2 * p
    p = 1.48572235717979e-05 + x2 * p
    p = 6.37261928875436e-04 + x2 * p
    p = 4.89352455891786e-03 + x2 * p
    p = x * p
    q = 1.19825839466702e-06
    q = 1.18534705686654e-04 + x2 * q
    q = 2.26843463243900e-03 + x2 * q
    q = 4.89352518554385e-03 + x2 * q
    return p / q


def _combine(part_ref, cnt_ref):
    s = part_ref[0] + part_ref[1]
    cnt = jnp.sum(cnt_ref[0] + cnt_ref[1], axis=1, keepdims=True)
    return s / jnp.maximum(cnt, 1.0)


def _dot(a, b):
    return jax.lax.dot(a, b, precision=jax.lax.Precision.HIGHEST,
                       preferred_element_type=jnp.float32)


def _proj(h, w_ref, b_ref, p_ref, q_ref):
    wt = w_ref[:D, :]
    wb = w_ref[D:, :]
    p_ref[...] = 2.0 * (_dot(h, wt - wb) + b_ref[...])
    q_ref[...] = 2.0 * _dot(h, wb)


def _prep_first_body(x_ref, w_ref, b_ref, p_ref, q_ref):
    _proj(x_ref[...], w_ref, b_ref, p_ref, q_ref)


def _prep_next_body(part_ref, cnt_ref, w_ref, b_ref, p_ref, q_ref):
    _proj(_combine(part_ref, cnt_ref), w_ref, b_ref, p_ref, q_ref)


def _dense_body(part_ref, cnt_ref, w0, b0, w1, b1, w2, b2, w3, b3, h_ref):
    h = _combine(part_ref, cnt_ref)
    for w_ref, b_ref in ((w0, b0), (w1, b1), (w2, b2), (w3, b3)):
        h = _tanh_f32(_dot(h, w_ref[...]) + b_ref[...])
    h_ref[...] = h


@functools.lru_cache(maxsize=None)
def _make_tc_kernels(n):
    grid = (n // RBLK,)
    row_spec = pl.BlockSpec((RBLK, D), lambda i: (i, 0))
    part_spec = pl.BlockSpec((2, RBLK, D), lambda i: (0, i, 0))
    cnt_spec = pl.BlockSpec((2, RBLK, L), lambda i: (0, i, 0))
    w_spec = pl.BlockSpec((2 * D, D), lambda i: (0, 0))
    wd_spec = pl.BlockSpec((D, D), lambda i: (0, 0))
    b_spec = pl.BlockSpec((1, D), lambda i: (0, 0))
    pq_out = [jax.ShapeDtypeStruct((n, D), jnp.float32)] * 2

    prep_first = pl.pallas_call(
        _prep_first_body, grid=grid,
        in_specs=[row_spec, w_spec, b_spec],
        out_specs=[row_spec, row_spec],
        out_shape=pq_out)
    prep_next = pl.pallas_call(
        _prep_next_body, grid=grid,
        in_specs=[part_spec, cnt_spec, w_spec, b_spec],
        out_specs=[row_spec, row_spec],
        out_shape=pq_out)
    dense = pl.pallas_call(
        _dense_body, grid=grid,
        in_specs=[part_spec, cnt_spec] + [wd_spec, b_spec] * 4,
        out_specs=row_spec,
        out_shape=jax.ShapeDtypeStruct((n, D), jnp.float32))
    return prep_first, prep_next, dense


@functools.lru_cache(maxsize=None)
def _make_sc_kernels(n, e):
    epw = e // NW
    nch = epw // K
    nchg = epw // KG
    assert nch % 4 == 0 and nchg % 2 == 0
    rpt = ((n + NS * 8 - 1) // (NS * 8)) * 8
    npad = rpt * NS
    mesh = plsc.VectorSubcoreMesh(core_axis_name="c", subcore_axis_name="s")
    sc_params = pltpu.CompilerParams(use_tc_tiling_on_sc=False,
                                     skip_device_barrier=True)

    def _wid_base():
        c = lax.axis_index("c")
        s = lax.axis_index("s")
        return c, s, s * NC + c

    @functools.partial(
        pl.kernel, mesh=mesh,
        out_type=jax.ShapeDtypeStruct((NC, npad, D), jnp.float32),
        compiler_params=sc_params,
        scratch_types=[
            pltpu.VMEM((nch, K), jnp.int32),
            pltpu.VMEM((nch, K), jnp.int32),
            pltpu.VMEM((K, D), jnp.float32),
            pltpu.VMEM((K, D), jnp.float32),
            pltpu.VMEM((K, D), jnp.float32),
            pltpu.VMEM((K, D), jnp.float32),
            pltpu.VMEM_SHARED((npad, D), jnp.float32),
            pltpu.SemaphoreType.DMA,
            pltpu.SemaphoreType.DMA,
            pltpu.SemaphoreType.DMA,
            pltpu.SemaphoreType.DMA,
            pltpu.SemaphoreType.DMA,
            pltpu.SemaphoreType.DMA,
            pltpu.SemaphoreType.DMA,
            pltpu.SemaphoreType.DMA,
            pltpu.SemaphoreType.DMA,
            pltpu.SemaphoreType.DMA,
            pltpu.SemaphoreType.DMA,
            pltpu.SemaphoreType.DMA,
        ])
    def edge_pass(p_hbm, q_hbm, src_hbm, dst_hbm, out_hbm,
                  idx_s, idx_d, b0, b1, b2, b3, acc,
                  ps0, ps1, ps2, ps3, qs0, qs1, qs2, qs3,
                  ss0, ss1, ss2, ss3):
        c, s, wid = _wid_base()
        row0 = s * rpt
        buf = (b0, b1, b2, b3)
        ps = (ps0, ps1, ps2, ps3)
        qs = (qs0, qs1, qs2, qs3)
        ss = (ss0, ss1, ss2, ss3)

        pltpu.sync_copy(src_hbm.at[wid], idx_s)
        pltpu.sync_copy(dst_hbm.at[wid], idx_d)

        def zrow(r, carry):
            for g in range(D // L):
                b3[r, pl.ds(g * L, L)] = jnp.zeros((L,), jnp.float32)
            return carry
        lax.fori_loop(0, K, zrow, 0)
        nz, rz = rpt // K, rpt % K
        zcopies = [pltpu.make_async_copy(
            b3.at[pl.ds(0, K)], acc.at[pl.ds(row0 + j * K, K)], ss3)
            for j in range(nz)]
        if rz:
            zcopies.append(pltpu.make_async_copy(
                b3.at[pl.ds(0, rz)], acc.at[pl.ds(row0 + nz * K, rz)], ss3))
        for cp in zcopies:
            cp.start()

        def pgather(ch, r):
            return pltpu.make_async_copy(p_hbm.at[idx_d.at[ch]], buf[r], ps[r])

        def qwait(ch, r):
            pltpu.make_async_copy(q_hbm.at[idx_s.at[ch]], buf[r], qs[r]).wait()

        def swait(ch, r):
            pltpu.make_async_copy(buf[r], acc.at[idx_d.at[ch]], ss[r]).wait()

        pgather(0, 0).start()
        pgather(1, 1).start()
        for cp in zcopies:
            cp.wait()
        plsc.subcore_barrier()
        pgather(0, 0).wait()
        pltpu.async_copy(q_hbm.at[idx_s.at[0]], b0, qs0, add=True)

        def body4(i, carry):
            for r in range(4):
                ch = 4 * i + r
                r1 = (r + 1) % 4
                r2 = (r + 2) % 4
                qwait(ch, r)
                @pl.when(ch >= 2)
                def _():
                    swait(ch - 2, r2)
                @pl.when(ch + 2 < nch)
                def _():
                    pgather(ch + 2, r2).start()
                @pl.when(ch + 1 < nch)
                def _():
                    pgather(ch + 1, r1).wait()
                    pltpu.async_copy(q_hbm.at[idx_s.at[ch + 1]], buf[r1],
                                     qs[r1], add=True)

                @plsc.parallel_loop(0, K, unroll=2)
                def _(rr):
                    for g in range(D // L):
                        v = buf[r][rr, pl.ds(g * L, L)]
                        ex = jnp.exp(v)
                        buf[r][rr, pl.ds(g * L, L)] = 1.0 - 2.0 / (ex + 1.0)
                pltpu.async_copy(buf[r], acc.at[idx_d.at[ch]], ss[r], add=True)
            return carry
        lax.fori_loop(0, nch // 4, body4, 0)
        swait(nch - 2, (nch - 2) % 4)
        swait(nch - 1, (nch - 1) % 4)
        plsc.subcore_barrier()

        ocopies = [pltpu.make_async_copy(
            acc.at[pl.ds(row0 + j * K, K)],
            out_hbm.at[c, pl.ds(row0 + j * K, K)], ss0)
            for j in range(nz)]
        if rz:
            ocopies.append(pltpu.make_async_copy(
                acc.at[pl.ds(row0 + nz * K, rz)],
                out_hbm.at[c, pl.ds(row0 + nz * K, rz)], ss0))
        for cp in ocopies:
            cp.start()
        for cp in ocopies:
            cp.wait()

    @functools.partial(
        pl.kernel, mesh=mesh,
        out_type=jax.ShapeDtypeStruct((NC, npad, L), jnp.float32),
        compiler_params=sc_params,
        scratch_types=[
            pltpu.VMEM((nchg, KG), jnp.int32),
            pltpu.VMEM((KG, L), jnp.float32),
            pltpu.VMEM_SHARED((npad, L), jnp.float32),
            pltpu.SemaphoreType.DMA,
        ])
    def count_pass(dst_hbm, out_hbm, idx_d, ones, cacc, csem):
        c, s, wid = _wid_base()
        row0 = s * rpt

        pltpu.sync_copy(dst_hbm.at[wid], idx_d)
        zrow16 = jnp.zeros((L,), jnp.float32)
        def zc(r, carry):
            ones[r, pl.ds(0, L)] = zrow16
            return carry
        lax.fori_loop(0, KG, zc, 0)
        for j in range(rpt // KG):
            pltpu.sync_copy(ones, cacc.at[pl.ds(row0 + j * KG, KG)])
        if rpt % KG:
            pltpu.sync_copy(ones.at[pl.ds(0, rpt % KG)],
                            cacc.at[pl.ds(row0 + (rpt // KG) * KG, rpt % KG)])
        onehot = jnp.where(lax.iota(jnp.int32, L) == 0,
                           jnp.float32(1.0), jnp.float32(0.0))
        def orow(r, carry):
            ones[r, pl.ds(0, L)] = onehot
            return carry
        lax.fori_loop(0, KG, orow, 0)
        plsc.subcore_barrier()

        def chunk(ch, carry):
            pltpu.async_copy(ones, cacc.at[idx_d.at[ch]], csem, add=True)
            return carry
        lax.fori_loop(0, nchg, chunk, 0)
        def drain(ch, carry):
            pltpu.make_async_copy(ones, cacc.at[idx_d.at[0]], csem).wait()
            return carry
        lax.fori_loop(0, nchg, drain, 0)
        plsc.subcore_barrier()

        for j in range(rpt // KG):
            pltpu.sync_copy(cacc.at[pl.ds(row0 + j * KG, KG)], ones)
            pltpu.sync_copy(ones, out_hbm.at[c, pl.ds(row0 + j * KG, KG)])
        rg = rpt % KG
        if rg:
            pltpu.sync_copy(cacc.at[pl.ds(row0 + (rpt // KG) * KG, rg)],
                            ones.at[pl.ds(0, rg)])
            pltpu.sync_copy(ones.at[pl.ds(0, rg)],
                            out_hbm.at[c, pl.ds(row0 + (rpt // KG) * KG, rg)])

    rph = n // NS

    @functools.partial(
        pl.kernel, mesh=mesh,
        out_type=jax.ShapeDtypeStruct((e, 2 * D), jnp.float32),
        compiler_params=sc_params,
        scratch_types=[
            pltpu.VMEM((nch, K), jnp.int32),
            pltpu.VMEM((nch, K), jnp.int32),
            pltpu.VMEM((K, D), jnp.float32),
            pltpu.VMEM((K, D), jnp.float32),
            pltpu.VMEM((K, D), jnp.float32),
            pltpu.VMEM((K, D), jnp.float32),
            pltpu.VMEM_SHARED((n, D), jnp.float32),
            pltpu.SemaphoreType.DMA,
            pltpu.SemaphoreType.DMA,
            pltpu.SemaphoreType.DMA,
            pltpu.SemaphoreType.DMA,
            pltpu.SemaphoreType.DMA,
            pltpu.SemaphoreType.DMA,
            pltpu.SemaphoreType.DMA,
            pltpu.SemaphoreType.DMA,
        ])
    def pair_gather(h_hbm, src_hbm, dst_hbm, out_hbm,
                    idx_s, idx_d, arow0, arow1, brow0, brow1, hsp,
                    ga0, ga1, gb0, gb1, wa0, wa1, wb0, wb1):
        c, s, wid = _wid_base()
        base = wid * epw
        arow = (arow0, arow1)
        brow = (brow0, brow1)
        ga = (ga0, ga1)
        gb = (gb0, gb1)
        wa = (wa0, wa1)
        wb = (wb0, wb1)

        pltpu.sync_copy(h_hbm.at[pl.ds(s * rph, rph)],
                        hsp.at[pl.ds(s * rph, rph)])
        pltpu.sync_copy(src_hbm.at[wid], idx_s)
        pltpu.sync_copy(dst_hbm.at[wid], idx_d)
        plsc.subcore_barrier()

        def gathers(ch, b):
            return (pltpu.make_async_copy(hsp.at[idx_s.at[ch]], arow[b], ga[b]),
                    pltpu.make_async_copy(hsp.at[idx_d.at[ch]], brow[b], gb[b]))

        def writes(ch, b):
            off = base + ch * K
            return (pltpu.make_async_copy(
                        arow[b], out_hbm.at[pl.ds(off, K), pl.ds(0, D)], wa[b]),
                    pltpu.make_async_copy(
                        brow[b], out_hbm.at[pl.ds(off, K), pl.ds(D, D)], wb[b]))

        for cp in gathers(0, 0):
            cp.start()

        def body2(i, carry):
            for b in range(2):
                ch = 2 * i + b
                nb = 1 - b
                for cp in gathers(ch, b):
                    cp.wait()
                @pl.when(ch + 1 < nch)
                def _():
                    for cp in gathers(ch + 1, nb):
                        cp.start()
                @pl.when(ch >= 2)
                def _():
                    for cp in writes(ch - 2, b):
                        cp.wait()
                for cp in writes(ch, b):
                    cp.start()
            return carry
        lax.fori_loop(0, nch // 2, body2, 0)
        for cp in writes(nch - 2, 0):
            cp.wait()
        for cp in writes(nch - 1, 1):
            cp.wait()

    return edge_pass, count_pass, pair_gather


def kernel(x, edge_index, Win, b_in, Wg0, bg0, Wg1, bg1, Wg2, bg2,
           Wd0, bd0, Wd1, bd1, Wd2, bd2, Wd3, bd3):
    n, d = x.shape
    e = edge_index.shape[1]
    assert d == D and n % RBLK == 0 and e % (NW * K) == 0 and e % (NW * KG) == 0

    prep_first, prep_next, dense = _make_tc_kernels(n)
    edge_pass, count_pass, pair_gather = _make_sc_kernels(n, e)

    epw = e // NW
    src = edge_index[0].reshape(NW, epw // K, K)
    dst = edge_index[1].reshape(NW, epw // K, K)
    dstg = edge_index[1].reshape(NW, epw // KG, KG)

    cnt = count_pass(dstg)
    p, q = prep_first(x, Win, b_in.reshape(1, D))
    part = edge_pass(p, q, src, dst)
    for w, b in ((Wg0, bg0), (Wg1, bg1), (Wg2, bg2)):
        p, q = prep_next(part, cnt, w, b.reshape(1, D))
        part = edge_pass(p, q, src, dst)
    h = dense(part, cnt, Wd0, bd0.reshape(1, D), Wd1, bd1.reshape(1, D),
              Wd2, bd2.reshape(1, D), Wd3, bd3.reshape(1, D))
    x_cat = pair_gather(h, src, dst)
    return (x_cat, edge_index)


# speedup vs baseline: 1.2529x; 1.2529x over previous
import functools

import jax
import jax.numpy as jnp
from jax import lax
from jax.experimental import pallas as pl
from jax.experimental.pallas import tpu as pltpu
from jax.experimental.pallas import tpu_sc as plsc

NC, NS, L = 2, 16, 16
NW = NC * NS

D = 128
K = 50
KP = 40
RBLK = 1000


def _tanh_f32(x):
    x = jnp.clip(x, -7.90531110763549805, 7.90531110763549805)
    x2 = x * x
    p = 2.00018790482477e-13 + x2 * -2.76076847742355e-16
    p = -8.60467152213735e-11 + x2 * p
    p = 5.12229709037114e-08 + x2 * p
    p = 1.48572235717979e-05 + x2 * p
    p = 6.37261928875436e-04 + x2 * p
    p = 4.89352455891786e-03 + x2 * p
    p = x * p
    q = 1.19825839466702e-06
    q = 1.18534705686654e-04 + x2 * q
    q = 2.26843463243900e-03 + x2 * q
    q = 4.89352518554385e-03 + x2 * q
    return p / q


def _combine(part_ref, cnt_ref):
    s = part_ref[0] + part_ref[1]
    cnt = jnp.sum(cnt_ref[0] + cnt_ref[1], axis=1, keepdims=True)
    return s / jnp.maximum(cnt, 1.0)


def _dot(a, b):
    return jax.lax.dot(a, b, precision=jax.lax.Precision.HIGHEST,
                       preferred_element_type=jnp.float32)


def _proj(h, w_ref, b_ref, p_ref, q_ref):
    wt = w_ref[:D, :]
    wb = w_ref[D:, :]
    p_ref[...] = 2.0 * (_dot(h, wt - wb) + b_ref[...])
    q_ref[...] = 2.0 * _dot(h, wb)


def _prep_first_body(x_ref, w_ref, b_ref, p_ref, q_ref):
    _proj(x_ref[...], w_ref, b_ref, p_ref, q_ref)


def _prep_next_body(part_ref, cnt_ref, w_ref, b_ref, p_ref, q_ref):
    _proj(_combine(part_ref, cnt_ref), w_ref, b_ref, p_ref, q_ref)


def _dense_body(part_ref, cnt_ref, w0, b0, w1, b1, w2, b2, w3, b3, h_ref):
    h = _combine(part_ref, cnt_ref)
    for w_ref, b_ref in ((w0, b0), (w1, b1), (w2, b2), (w3, b3)):
        h = _tanh_f32(_dot(h, w_ref[...]) + b_ref[...])
    h_ref[...] = h


@functools.lru_cache(maxsize=None)
def _make_tc_kernels(n):
    grid = (n // RBLK,)
    row_spec = pl.BlockSpec((RBLK, D), lambda i: (i, 0))
    part_spec = pl.BlockSpec((2, RBLK, D), lambda i: (0, i, 0))
    cnt_spec = pl.BlockSpec((2, RBLK, L), lambda i: (0, i, 0))
    w_spec = pl.BlockSpec((2 * D, D), lambda i: (0, 0))
    wd_spec = pl.BlockSpec((D, D), lambda i: (0, 0))
    b_spec = pl.BlockSpec((1, D), lambda i: (0, 0))
    pq_out = [jax.ShapeDtypeStruct((n, D), jnp.float32)] * 2

    prep_first = pl.pallas_call(
        _prep_first_body, grid=grid,
        in_specs=[row_spec, w_spec, b_spec],
        out_specs=[row_spec, row_spec],
        out_shape=pq_out)
    prep_next = pl.pallas_call(
        _prep_next_body, grid=grid,
        in_specs=[part_spec, cnt_spec, w_spec, b_spec],
        out_specs=[row_spec, row_spec],
        out_shape=pq_out)
    dense = pl.pallas_call(
        _dense_body, grid=grid,
        in_specs=[part_spec, cnt_spec] + [wd_spec, b_spec] * 4,
        out_specs=row_spec,
        out_shape=jax.ShapeDtypeStruct((n, D), jnp.float32))
    return prep_first, prep_next, dense


@functools.lru_cache(maxsize=None)
def _make_sc_kernels(n, e):
    epw = e // NW
    nch = epw // K
    assert nch % 4 == 0
    rpt = ((n + NS * 8 - 1) // (NS * 8)) * 8
    npad = rpt * NS
    mesh = plsc.VectorSubcoreMesh(core_axis_name="c", subcore_axis_name="s")
    sc_params = pltpu.CompilerParams(use_tc_tiling_on_sc=False)

    def _wid_base():
        c = lax.axis_index("c")
        s = lax.axis_index("s")
        return c, s, s * NC + c

    @functools.partial(
        pl.kernel, mesh=mesh,
        out_type=jax.ShapeDtypeStruct((NC, npad, D), jnp.float32),
        compiler_params=sc_params,
        scratch_types=[
            pltpu.VMEM((nch, K), jnp.int32),
            pltpu.VMEM((nch, K), jnp.int32),
            pltpu.VMEM((K, D), jnp.float32),
            pltpu.VMEM((K, D), jnp.float32),
            pltpu.VMEM((K, D), jnp.float32),
            pltpu.VMEM((K, D), jnp.float32),
            pltpu.VMEM_SHARED((npad, D), jnp.float32),
            pltpu.SemaphoreType.DMA,
            pltpu.SemaphoreType.DMA,
            pltpu.SemaphoreType.DMA,
            pltpu.SemaphoreType.DMA,
            pltpu.SemaphoreType.DMA,
            pltpu.SemaphoreType.DMA,
            pltpu.SemaphoreType.DMA,
            pltpu.SemaphoreType.DMA,
            pltpu.SemaphoreType.DMA,
            pltpu.SemaphoreType.DMA,
            pltpu.SemaphoreType.DMA,
            pltpu.SemaphoreType.DMA,
        ])
    def edge_pass(p_hbm, q_hbm, src_hbm, dst_hbm, out_hbm,
                  idx_s, idx_d, b0, b1, b2, b3, acc,
                  ps0, ps1, ps2, ps3, qs0, qs1, qs2, qs3,
                  ss0, ss1, ss2, ss3):
        c, s, wid = _wid_base()
        row0 = s * rpt
        buf = (b0, b1, b2, b3)
        ps = (ps0, ps1, ps2, ps3)
        qs = (qs0, qs1, qs2, qs3)
        ss = (ss0, ss1, ss2, ss3)

        pltpu.sync_copy(src_hbm.at[wid], idx_s)
        pltpu.sync_copy(dst_hbm.at[wid], idx_d)

        def zrow(r, carry):
            for g in range(D // L):
                b3[r, pl.ds(g * L, L)] = jnp.zeros((L,), jnp.float32)
            return carry
        lax.fori_loop(0, K, zrow, 0)
        nz, rz = rpt // K, rpt % K
        zcopies = [pltpu.make_async_copy(
            b3.at[pl.ds(0, K)], acc.at[pl.ds(row0 + j * K, K)], ss3)
            for j in range(nz)]
        if rz:
            zcopies.append(pltpu.make_async_copy(
                b3.at[pl.ds(0, rz)], acc.at[pl.ds(row0 + nz * K, rz)], ss3))
        for cp in zcopies:
            cp.start()

        def pgather(ch, r):
            return pltpu.make_async_copy(p_hbm.at[idx_d.at[ch]], buf[r], ps[r])

        def qwait(ch, r):
            pltpu.make_async_copy(q_hbm.at[idx_s.at[ch]], buf[r], qs[r]).wait()

        def swait(ch, r):
            pltpu.make_async_copy(buf[r], acc.at[idx_d.at[ch]], ss[r]).wait()

        pgather(0, 0).start()
        pgather(1, 1).start()
        for cp in zcopies:
            cp.wait()
        plsc.subcore_barrier()
        pgather(0, 0).wait()
        pltpu.async_copy(q_hbm.at[idx_s.at[0]], b0, qs0, add=True)

        def body4(i, carry):
            for r in range(4):
                ch = 4 * i + r
                r1 = (r + 1) % 4
                r2 = (r + 2) % 4
                qwait(ch, r)
                @pl.when(ch >= 2)
                def _():
                    swait(ch - 2, r2)
                @pl.when(ch + 2 < nch)
                def _():
                    pgather(ch + 2, r2).start()
                @pl.when(ch + 1 < nch)
                def _():
                    pgather(ch + 1, r1).wait()
                    pltpu.async_copy(q_hbm.at[idx_s.at[ch + 1]], buf[r1],
                                     qs[r1], add=True)

                @plsc.parallel_loop(0, K, unroll=2)
                def _(rr):
                    for g in range(D // L):
                        v = buf[r][rr, pl.ds(g * L, L)]
                        ex = jnp.exp(v)
                        buf[r][rr, pl.ds(g * L, L)] = 1.0 - 2.0 / (ex + 1.0)
                pltpu.async_copy(buf[r], acc.at[idx_d.at[ch]], ss[r], add=True)
            return carry
        lax.fori_loop(0, nch // 4, body4, 0)
        swait(nch - 2, (nch - 2) % 4)
        swait(nch - 1, (nch - 1) % 4)
        plsc.subcore_barrier()

        ocopies = [pltpu.make_async_copy(
            acc.at[pl.ds(row0 + j * K, K)],
            out_hbm.at[c, pl.ds(row0 + j * K, K)], ss0)
            for j in range(nz)]
        if rz:
            ocopies.append(pltpu.make_async_copy(
                acc.at[pl.ds(row0 + nz * K, rz)],
                out_hbm.at[c, pl.ds(row0 + nz * K, rz)], ss0))
        for cp in ocopies:
            cp.start()
        for cp in ocopies:
            cp.wait()

    @functools.partial(
        pl.kernel, mesh=mesh,
        out_type=jax.ShapeDtypeStruct((NC, npad, L), jnp.float32),
        compiler_params=sc_params,
        scratch_types=[
            pltpu.VMEM((nch, K), jnp.int32),
            pltpu.VMEM((K, L), jnp.float32),
            pltpu.VMEM_SHARED((npad, L), jnp.float32),
            pltpu.SemaphoreType.DMA,
        ])
    def count_pass(dst_hbm, out_hbm, idx_d, ones, cacc, csem):
        c, s, wid = _wid_base()
        row0 = s * rpt

        pltpu.sync_copy(dst_hbm.at[wid], idx_d)
        zrow16 = jnp.zeros((L,), jnp.float32)
        def zc(r, carry):
            ones[r, pl.ds(0, L)] = zrow16
            return carry
        lax.fori_loop(0, K, zc, 0)
        for j in range(rpt // K):
            pltpu.sync_copy(ones, cacc.at[pl.ds(row0 + j * K, K)])
        if rpt % K:
            pltpu.sync_copy(ones.at[pl.ds(0, rpt % K)],
                            cacc.at[pl.ds(row0 + (rpt // K) * K, rpt % K)])
        onehot = jnp.where(lax.iota(jnp.int32, L) == 0,
                           jnp.float32(1.0), jnp.float32(0.0))
        def orow(r, carry):
            ones[r, pl.ds(0, L)] = onehot
            return carry
        lax.fori_loop(0, K, orow, 0)
        plsc.subcore_barrier()

        def chunk(ch, carry):
            pltpu.async_copy(ones, cacc.at[idx_d.at[ch]], csem, add=True)
            return carry
        lax.fori_loop(0, nch, chunk, 0)
        def drain(ch, carry):
            pltpu.make_async_copy(ones, cacc.at[idx_d.at[0]], csem).wait()
            return carry
        lax.fori_loop(0, nch, drain, 0)
        plsc.subcore_barrier()

        for j in range(rpt // K):
            pltpu.sync_copy(cacc.at[pl.ds(row0 + j * K, K)], ones)
            pltpu.sync_copy(ones, out_hbm.at[c, pl.ds(row0 + j * K, K)])
        rg = rpt % K
        if rg:
            pltpu.sync_copy(cacc.at[pl.ds(row0 + (rpt // K) * K, rg)],
                            ones.at[pl.ds(0, rg)])
            pltpu.sync_copy(ones.at[pl.ds(0, rg)],
                            out_hbm.at[c, pl.ds(row0 + (rpt // K) * K, rg)])

    nchp = epw // KP
    assert nchp % 2 == 0
    lasth = n - (NS - 1) * rpt

    @functools.partial(
        pl.kernel, mesh=mesh,
        out_type=jax.ShapeDtypeStruct((e, 2 * D), jnp.float32),
        scratch_types=[
            pltpu.VMEM((KP,), jnp.int32),
            pltpu.VMEM((KP,), jnp.int32),
            pltpu.VMEM((KP,), jnp.int32),
            pltpu.VMEM((KP,), jnp.int32),
            pltpu.VMEM((KP, D), jnp.float32),
            pltpu.VMEM((KP, D), jnp.float32),
            pltpu.VMEM((KP, D), jnp.float32),
            pltpu.VMEM((KP, D), jnp.float32),
            pltpu.VMEM_SHARED((n, D), jnp.float32),
            pltpu.SemaphoreType.DMA,
            pltpu.SemaphoreType.DMA,
            pltpu.SemaphoreType.DMA,
            pltpu.SemaphoreType.DMA,
            pltpu.SemaphoreType.DMA,
            pltpu.SemaphoreType.DMA,
            pltpu.SemaphoreType.DMA,
            pltpu.SemaphoreType.DMA,
            pltpu.SemaphoreType.DMA,
            pltpu.SemaphoreType.DMA,
            pltpu.SemaphoreType.DMA,
            pltpu.SemaphoreType.DMA,
        ])
    def pair_gather(h_hbm, src_hbm, dst_hbm, out_hbm,
                    isb0, isb1, idb0, idb1, arow0, arow1, brow0, brow1, hsp,
                    lis0, lis1, lid0, lid1, ga0, ga1, gb0, gb1,
                    wa0, wa1, wb0, wb1):
        c, s, wid = _wid_base()
        base = wid * epw
        isb = (isb0, isb1)
        idb = (idb0, idb1)
        arow = (arow0, arow1)
        brow = (brow0, brow1)
        lis = (lis0, lis1)
        lid = (lid0, lid1)
        ga = (ga0, ga1)
        gb = (gb0, gb1)
        wa = (wa0, wa1)
        wb = (wb0, wb1)

        @pl.when(s < NS - 1)
        def _():
            pltpu.sync_copy(h_hbm.at[pl.ds(s * rpt, rpt)],
                            hsp.at[pl.ds(s * rpt, rpt)])
        @pl.when(s == NS - 1)
        def _():
            pltpu.sync_copy(h_hbm.at[pl.ds((NS - 1) * rpt, lasth)],
                            hsp.at[pl.ds((NS - 1) * rpt, lasth)])
        plsc.subcore_barrier()

        def idxload(ch, b):
            return (pltpu.make_async_copy(
                        src_hbm.at[pl.ds(base + ch * KP, KP)], isb[b], lis[b]),
                    pltpu.make_async_copy(
                        dst_hbm.at[pl.ds(base + ch * KP, KP)], idb[b], lid[b]))

        def gathers(ch, b):
            return (pltpu.make_async_copy(hsp.at[isb[b]], arow[b], ga[b]),
                    pltpu.make_async_copy(hsp.at[idb[b]], brow[b], gb[b]))

        def writes(ch, b):
            off = base + ch * KP
            return (pltpu.make_async_copy(
                        arow[b], out_hbm.at[pl.ds(off, KP), pl.ds(0, D)], wa[b]),
                    pltpu.make_async_copy(
                        brow[b], out_hbm.at[pl.ds(off, KP), pl.ds(D, D)], wb[b]))

        for cp in idxload(0, 0):
            cp.start()
        for cp in idxload(0, 0):
            cp.wait()
        for cp in idxload(1, 1):
            cp.start()
        for cp in gathers(0, 0):
            cp.start()

        def body2(i, carry):
            for b in range(2):
                ch = 2 * i + b
                nb = 1 - b
                for cp in gathers(ch, b):
                    cp.wait()
                @pl.when(ch >= 1)
                def _():
                    for cp in writes(ch - 1, nb):
                        cp.wait()
                @pl.when(ch + 1 < nchp)
                def _():
                    for cp in idxload(ch + 1, nb):
                        cp.wait()
                    for cp in gathers(ch + 1, nb):
                        cp.start()
                @pl.when(ch + 2 < nchp)
                def _():
                    for cp in idxload(ch + 2, b):
                        cp.start()
                for cp in writes(ch, b):
                    cp.start()
            return carry
        lax.fori_loop(0, nchp // 2, body2, 0)
        for cp in writes(nchp - 1, 1):
            cp.wait()

    return edge_pass, count_pass, pair_gather


def kernel(x, edge_index, Win, b_in, Wg0, bg0, Wg1, bg1, Wg2, bg2,
           Wd0, bd0, Wd1, bd1, Wd2, bd2, Wd3, bd3):
    n, d = x.shape
    e = edge_index.shape[1]
    assert d == D and n % RBLK == 0 and e % (NW * K) == 0 and e % (NW * KP) == 0

    prep_first, prep_next, dense = _make_tc_kernels(n)
    edge_pass, count_pass, pair_gather = _make_sc_kernels(n, e)

    epw = e // NW
    src = edge_index[0].reshape(NW, epw // K, K)
    dst = edge_index[1].reshape(NW, epw // K, K)

    cnt = count_pass(dst)
    p, q = prep_first(x, Win, b_in.reshape(1, D))
    part = edge_pass(p, q, src, dst)
    for w, b in ((Wg0, bg0), (Wg1, bg1), (Wg2, bg2)):
        p, q = prep_next(part, cnt, w, b.reshape(1, D))
        part = edge_pass(p, q, src, dst)
    h = dense(part, cnt, Wd0, bd0.reshape(1, D), Wd1, bd1.reshape(1, D),
              Wd2, bd2.reshape(1, D), Wd3, bd3.reshape(1, D))
    x_cat = pair_gather(h, edge_index[0], edge_index[1])
    return (x_cat, edge_index)


# speedup vs baseline: 1.2548x; 1.0015x over previous
import functools

import jax
import jax.numpy as jnp
from jax import lax
from jax.experimental import pallas as pl
from jax.experimental.pallas import tpu as pltpu
from jax.experimental.pallas import tpu_sc as plsc

NC, NS, L = 2, 16, 16
NW = NC * NS

D = 128
K = 50
KP = 40
RBLK = 1000


def _tanh_f32(x):
    x = jnp.clip(x, -7.90531110763549805, 7.90531110763549805)
    x2 = x * x
    p = 2.00018790482477e-13 + x2 * -2.76076847742355e-16
    p = -8.60467152213735e-11 + x2 * p
    p = 5.12229709037114e-08 + x2 * p
    p = 1.48572235717979e-05 + x2 * p
    p = 6.37261928875436e-04 + x2 * p
    p = 4.89352455891786e-03 + x2 * p
    p = x * p
    q = 1.19825839466702e-06
    q = 1.18534705686654e-04 + x2 * q
    q = 2.26843463243900e-03 + x2 * q
    q = 4.89352518554385e-03 + x2 * q
    return p / q


def _combine(part_ref, cnt_ref):
    s = part_ref[0] + part_ref[1]
    cnt = jnp.sum(cnt_ref[0] + cnt_ref[1], axis=1, keepdims=True)
    return s / jnp.maximum(cnt, 1.0)


def _dot(a, b):
    return jax.lax.dot(a, b, precision=jax.lax.Precision.HIGHEST,
                       preferred_element_type=jnp.float32)


def _proj(h, w_ref, b_ref, p_ref, q_ref):
    wt = w_ref[:D, :]
    wb = w_ref[D:, :]
    p_ref[...] = 2.0 * (_dot(h, wt - wb) + b_ref[...])
    q_ref[...] = 2.0 * _dot(h, wb)


def _prep_first_body(x_ref, w_ref, b_ref, p_ref, q_ref):
    _proj(x_ref[...], w_ref, b_ref, p_ref, q_ref)


def _prep_next_body(part_ref, cnt_ref, w_ref, b_ref, p_ref, q_ref):
    _proj(_combine(part_ref, cnt_ref), w_ref, b_ref, p_ref, q_ref)


def _dense_body(part_ref, cnt_ref, w0, b0, w1, b1, w2, b2, w3, b3, h_ref):
    h = _combine(part_ref, cnt_ref)
    for w_ref, b_ref in ((w0, b0), (w1, b1), (w2, b2), (w3, b3)):
        h = _tanh_f32(_dot(h, w_ref[...]) + b_ref[...])
    h_ref[...] = h


@functools.lru_cache(maxsize=None)
def _make_tc_kernels(n):
    grid = (n // RBLK,)
    row_spec = pl.BlockSpec((RBLK, D), lambda i: (i, 0))
    part_spec = pl.BlockSpec((2, RBLK, D), lambda i: (0, i, 0))
    cnt_spec = pl.BlockSpec((2, RBLK, L), lambda i: (0, i, 0))
    w_spec = pl.BlockSpec((2 * D, D), lambda i: (0, 0))
    wd_spec = pl.BlockSpec((D, D), lambda i: (0, 0))
    b_spec = pl.BlockSpec((1, D), lambda i: (0, 0))
    pq_out = [jax.ShapeDtypeStruct((n, D), jnp.float32)] * 2

    prep_first = pl.pallas_call(
        _prep_first_body, grid=grid,
        in_specs=[row_spec, w_spec, b_spec],
        out_specs=[row_spec, row_spec],
        out_shape=pq_out)
    prep_next = pl.pallas_call(
        _prep_next_body, grid=grid,
        in_specs=[part_spec, cnt_spec, w_spec, b_spec],
        out_specs=[row_spec, row_spec],
        out_shape=pq_out)
    dense = pl.pallas_call(
        _dense_body, grid=grid,
        in_specs=[part_spec, cnt_spec] + [wd_spec, b_spec] * 4,
        out_specs=row_spec,
        out_shape=jax.ShapeDtypeStruct((n, D), jnp.float32))
    return prep_first, prep_next, dense


@functools.lru_cache(maxsize=None)
def _make_sc_kernels(n, e):
    epw = e // NW
    nch = epw // K
    assert nch % 4 == 0
    rpt = ((n + NS * 8 - 1) // (NS * 8)) * 8
    npad = rpt * NS
    mesh = plsc.VectorSubcoreMesh(core_axis_name="c", subcore_axis_name="s")
    sc_params = pltpu.CompilerParams(use_tc_tiling_on_sc=False)

    def _wid_base():
        c = lax.axis_index("c")
        s = lax.axis_index("s")
        return c, s, s * NC + c

    @functools.partial(
        pl.kernel, mesh=mesh,
        out_type=jax.ShapeDtypeStruct((NC, npad, D), jnp.float32),
        compiler_params=sc_params,
        scratch_types=[
            pltpu.VMEM((nch, K), jnp.int32),
            pltpu.VMEM((nch, K), jnp.int32),
            pltpu.VMEM((K, D), jnp.float32),
            pltpu.VMEM((K, D), jnp.float32),
            pltpu.VMEM((K, D), jnp.float32),
            pltpu.VMEM((K, D), jnp.float32),
            pltpu.VMEM_SHARED((npad, D), jnp.float32),
            pltpu.SemaphoreType.DMA,
            pltpu.SemaphoreType.DMA,
            pltpu.SemaphoreType.DMA,
            pltpu.SemaphoreType.DMA,
            pltpu.SemaphoreType.DMA,
            pltpu.SemaphoreType.DMA,
            pltpu.SemaphoreType.DMA,
            pltpu.SemaphoreType.DMA,
            pltpu.SemaphoreType.DMA,
            pltpu.SemaphoreType.DMA,
            pltpu.SemaphoreType.DMA,
            pltpu.SemaphoreType.DMA,
        ])
    def edge_pass(p_hbm, q_hbm, src_hbm, dst_hbm, out_hbm,
                  idx_s, idx_d, b0, b1, b2, b3, acc,
                  ps0, ps1, ps2, ps3, qs0, qs1, qs2, qs3,
                  ss0, ss1, ss2, ss3):
        c, s, wid = _wid_base()
        row0 = s * rpt
        buf = (b0, b1, b2, b3)
        ps = (ps0, ps1, ps2, ps3)
        qs = (qs0, qs1, qs2, qs3)
        ss = (ss0, ss1, ss2, ss3)

        pltpu.sync_copy(src_hbm.at[wid], idx_s)
        pltpu.sync_copy(dst_hbm.at[wid], idx_d)

        def zrow(r, carry):
            for g in range(D // L):
                b3[r, pl.ds(g * L, L)] = jnp.zeros((L,), jnp.float32)
            return carry
        lax.fori_loop(0, K, zrow, 0)
        nz, rz = rpt // K, rpt % K
        zcopies = [pltpu.make_async_copy(
            b3.at[pl.ds(0, K)], acc.at[pl.ds(row0 + j * K, K)], ss3)
            for j in range(nz)]
        if rz:
            zcopies.append(pltpu.make_async_copy(
                b3.at[pl.ds(0, rz)], acc.at[pl.ds(row0 + nz * K, rz)], ss3))
        for cp in zcopies:
            cp.start()

        def pgather(ch, r):
            return pltpu.make_async_copy(p_hbm.at[idx_d.at[ch]], buf[r], ps[r])

        def qwait(ch, r):
            pltpu.make_async_copy(q_hbm.at[idx_s.at[ch]], buf[r], qs[r]).wait()

        def swait(ch, r):
            pltpu.make_async_copy(buf[r], acc.at[idx_d.at[ch]], ss[r]).wait()

        pgather(0, 0).start()
        pgather(1, 1).start()
        for cp in zcopies:
            cp.wait()
        plsc.subcore_barrier()
        pgather(0, 0).wait()
        pltpu.async_copy(q_hbm.at[idx_s.at[0]], b0, qs0, add=True)

        def body4(i, carry):
            for r in range(4):
                ch = 4 * i + r
                r1 = (r + 1) % 4
                r2 = (r + 2) % 4
                @pl.when(ch + 1 < nch)
                def _():
                    pgather(ch + 1, r1).wait()
                    pltpu.async_copy(q_hbm.at[idx_s.at[ch + 1]], buf[r1],
                                     qs[r1], add=True)
                qwait(ch, r)
                @pl.when(ch >= 2)
                def _():
                    swait(ch - 2, r2)
                @pl.when(ch + 2 < nch)
                def _():
                    pgather(ch + 2, r2).start()

                @plsc.parallel_loop(0, K, unroll=2)
                def _(rr):
                    for g in range(D // L):
                        v = buf[r][rr, pl.ds(g * L, L)]
                        ex = jnp.exp(v)
                        buf[r][rr, pl.ds(g * L, L)] = 1.0 - 2.0 / (ex + 1.0)
                pltpu.async_copy(buf[r], acc.at[idx_d.at[ch]], ss[r], add=True)
            return carry
        lax.fori_loop(0, nch // 4, body4, 0)
        swait(nch - 2, (nch - 2) % 4)
        swait(nch - 1, (nch - 1) % 4)
        plsc.subcore_barrier()

        ocopies = [pltpu.make_async_copy(
            acc.at[pl.ds(row0 + j * K, K)],
            out_hbm.at[c, pl.ds(row0 + j * K, K)], ss0)
            for j in range(nz)]
        if rz:
            ocopies.append(pltpu.make_async_copy(
                acc.at[pl.ds(row0 + nz * K, rz)],
                out_hbm.at[c, pl.ds(row0 + nz * K, rz)], ss0))
        for cp in ocopies:
            cp.start()
        for cp in ocopies:
            cp.wait()

    @functools.partial(
        pl.kernel, mesh=mesh,
        out_type=jax.ShapeDtypeStruct((NC, npad, L), jnp.float32),
        compiler_params=sc_params,
        scratch_types=[
            pltpu.VMEM((nch, K), jnp.int32),
            pltpu.VMEM((K, L), jnp.float32),
            pltpu.VMEM_SHARED((npad, L), jnp.float32),
            pltpu.SemaphoreType.DMA,
        ])
    def count_pass(dst_hbm, out_hbm, idx_d, ones, cacc, csem):
        c, s, wid = _wid_base()
        row0 = s * rpt

        pltpu.sync_copy(dst_hbm.at[wid], idx_d)
        zrow16 = jnp.zeros((L,), jnp.float32)
        def zc(r, carry):
            ones[r, pl.ds(0, L)] = zrow16
            return carry
        lax.fori_loop(0, K, zc, 0)
        for j in range(rpt // K):
            pltpu.sync_copy(ones, cacc.at[pl.ds(row0 + j * K, K)])
        if rpt % K:
            pltpu.sync_copy(ones.at[pl.ds(0, rpt % K)],
                            cacc.at[pl.ds(row0 + (rpt // K) * K, rpt % K)])
        onehot = jnp.where(lax.iota(jnp.int32, L) == 0,
                           jnp.float32(1.0), jnp.float32(0.0))
        def orow(r, carry):
            ones[r, pl.ds(0, L)] = onehot
            return carry
        lax.fori_loop(0, K, orow, 0)
        plsc.subcore_barrier()

        def chunk(ch, carry):
            pltpu.async_copy(ones, cacc.at[idx_d.at[ch]], csem, add=True)
            return carry
        lax.fori_loop(0, nch, chunk, 0)
        def drain(ch, carry):
            pltpu.make_async_copy(ones, cacc.at[idx_d.at[0]], csem).wait()
            return carry
        lax.fori_loop(0, nch, drain, 0)
        plsc.subcore_barrier()

        for j in range(rpt // K):
            pltpu.sync_copy(cacc.at[pl.ds(row0 + j * K, K)], ones)
            pltpu.sync_copy(ones, out_hbm.at[c, pl.ds(row0 + j * K, K)])
        rg = rpt % K
        if rg:
            pltpu.sync_copy(cacc.at[pl.ds(row0 + (rpt // K) * K, rg)],
                            ones.at[pl.ds(0, rg)])
            pltpu.sync_copy(ones.at[pl.ds(0, rg)],
                            out_hbm.at[c, pl.ds(row0 + (rpt // K) * K, rg)])

    nchp = epw // KP
    assert nchp % 2 == 0
    lasth = n - (NS - 1) * rpt

    @functools.partial(
        pl.kernel, mesh=mesh,
        out_type=jax.ShapeDtypeStruct((e, 2 * D), jnp.float32),
        scratch_types=[
            pltpu.VMEM((KP,), jnp.int32),
            pltpu.VMEM((KP,), jnp.int32),
            pltpu.VMEM((KP,), jnp.int32),
            pltpu.VMEM((KP,), jnp.int32),
            pltpu.VMEM((KP, D), jnp.float32),
            pltpu.VMEM((KP, D), jnp.float32),
            pltpu.VMEM((KP, D), jnp.float32),
            pltpu.VMEM((KP, D), jnp.float32),
            pltpu.VMEM_SHARED((n, D), jnp.float32),
            pltpu.SemaphoreType.DMA,
            pltpu.SemaphoreType.DMA,
            pltpu.SemaphoreType.DMA,
            pltpu.SemaphoreType.DMA,
            pltpu.SemaphoreType.DMA,
            pltpu.SemaphoreType.DMA,
            pltpu.SemaphoreType.DMA,
            pltpu.SemaphoreType.DMA,
            pltpu.SemaphoreType.DMA,
            pltpu.SemaphoreType.DMA,
            pltpu.SemaphoreType.DMA,
            pltpu.SemaphoreType.DMA,
        ])
    def pair_gather(h_hbm, src_hbm, dst_hbm, out_hbm,
                    isb0, isb1, idb0, idb1, arow0, arow1, brow0, brow1, hsp,
                    lis0, lis1, lid0, lid1, ga0, ga1, gb0, gb1,
                    wa0, wa1, wb0, wb1):
        c, s, wid = _wid_base()
        base = wid * epw
        isb = (isb0, isb1)
        idb = (idb0, idb1)
        arow = (arow0, arow1)
        brow = (brow0, brow1)
        lis = (lis0, lis1)
        lid = (lid0, lid1)
        ga = (ga0, ga1)
        gb = (gb0, gb1)
        wa = (wa0, wa1)
        wb = (wb0, wb1)

        @pl.when(s < NS - 1)
        def _():
            pltpu.sync_copy(h_hbm.at[pl.ds(s * rpt, rpt)],
                            hsp.at[pl.ds(s * rpt, rpt)])
        @pl.when(s == NS - 1)
        def _():
            pltpu.sync_copy(h_hbm.at[pl.ds((NS - 1) * rpt, lasth)],
                            hsp.at[pl.ds((NS - 1) * rpt, lasth)])
        plsc.subcore_barrier()

        def idxload(ch, b):
            return (pltpu.make_async_copy(
                        src_hbm.at[pl.ds(base + ch * KP, KP)], isb[b], lis[b]),
                    pltpu.make_async_copy(
                        dst_hbm.at[pl.ds(base + ch * KP, KP)], idb[b], lid[b]))

        def gathers(ch, b):
            return (pltpu.make_async_copy(hsp.at[isb[b]], arow[b], ga[b]),
                    pltpu.make_async_copy(hsp.at[idb[b]], brow[b], gb[b]))

        def writes(ch, b):
            off = base + ch * KP
            return (pltpu.make_async_copy(
                        arow[b], out_hbm.at[pl.ds(off, KP), pl.ds(0, D)], wa[b]),
                    pltpu.make_async_copy(
                        brow[b], out_hbm.at[pl.ds(off, KP), pl.ds(D, D)], wb[b]))

        for cp in idxload(0, 0):
            cp.start()
        for cp in idxload(0, 0):
            cp.wait()
        for cp in idxload(1, 1):
            cp.start()
        for cp in gathers(0, 0):
            cp.start()

        def body2(i, carry):
            for b in range(2):
                ch = 2 * i + b
                nb = 1 - b
                for cp in gathers(ch, b):
                    cp.wait()
                @pl.when(ch >= 1)
                def _():
                    for cp in writes(ch - 1, nb):
                        cp.wait()
                @pl.when(ch + 1 < nchp)
                def _():
                    for cp in idxload(ch + 1, nb):
                        cp.wait()
                    for cp in gathers(ch + 1, nb):
                        cp.start()
                @pl.when(ch + 2 < nchp)
                def _():
                    for cp in idxload(ch + 2, b):
                        cp.start()
                for cp in writes(ch, b):
                    cp.start()
            return carry
        lax.fori_loop(0, nchp // 2, body2, 0)
        for cp in writes(nchp - 1, 1):
            cp.wait()

    return edge_pass, count_pass, pair_gather


def kernel(x, edge_index, Win, b_in, Wg0, bg0, Wg1, bg1, Wg2, bg2,
           Wd0, bd0, Wd1, bd1, Wd2, bd2, Wd3, bd3):
    n, d = x.shape
    e = edge_index.shape[1]
    assert d == D and n % RBLK == 0 and e % (NW * K) == 0 and e % (NW * KP) == 0

    prep_first, prep_next, dense = _make_tc_kernels(n)
    edge_pass, count_pass, pair_gather = _make_sc_kernels(n, e)

    epw = e // NW
    src = edge_index[0].reshape(NW, epw // K, K)
    dst = edge_index[1].reshape(NW, epw // K, K)

    cnt = count_pass(dst)
    p, q = prep_first(x, Win, b_in.reshape(1, D))
    part = edge_pass(p, q, src, dst)
    for w, b in ((Wg0, bg0), (Wg1, bg1), (Wg2, bg2)):
        p, q = prep_next(part, cnt, w, b.reshape(1, D))
        part = edge_pass(p, q, src, dst)
    h = dense(part, cnt, Wd0, bd0.reshape(1, D), Wd1, bd1.reshape(1, D),
              Wd2, bd2.reshape(1, D), Wd3, bd3.reshape(1, D))
    x_cat = pair_gather(h, edge_index[0], edge_index[1])
    return (x_cat, edge_index)
